# baseline (speedup 1.0000x reference)
import functools

import jax
import jax.numpy as jnp
from jax import lax
from jax.experimental import pallas as pl
from jax.experimental.pallas import tpu as pltpu, tpu_sc as plsc

NUM_SENTENCES = 1000000
EMBED_DIM = 64
BATCH = 16384

_info = plsc.get_sparse_core_info()
_NC, _NS = _info.num_cores, _info.num_subcores
_NW = _NC * _NS
_B_PER_W = BATCH // _NW
_IDX_CHUNK = 128

_mesh = plsc.VectorSubcoreMesh(core_axis_name="c", subcore_axis_name="s")


@functools.partial(
    pl.kernel,
    mesh=_mesh,
    out_type=(
        jax.ShapeDtypeStruct((BATCH, EMBED_DIM), jnp.float32),
        jax.ShapeDtypeStruct((BATCH, 1, EMBED_DIM), jnp.float32),
    ),
    scratch_types=[
        pltpu.VMEM((_B_PER_W,), jnp.int32),
        pltpu.VMEM((_IDX_CHUNK, 1), jnp.int32),
        pltpu.VMEM((_B_PER_W, EMBED_DIM), jnp.float32),
        pltpu.SemaphoreType.DMA,
        pltpu.SemaphoreType.DMA,
    ],
    compiler_params=pltpu.CompilerParams(needs_layout_passes=False),
)
def _gather_rows(table_hbm, idx_hbm, out_hbm, out2_hbm, idx_v, idxc_v, rows_v, sem, sem2):
    wid = lax.axis_index("s") * _NC + lax.axis_index("c")
    base = wid * _B_PER_W

    lane = lax.iota(jnp.int32, 16)
    zero16 = jnp.zeros((16,), jnp.int32)
    for c in range(_B_PER_W // _IDX_CHUNK):
        pltpu.sync_copy(
            idx_hbm.at[pl.ds(base + c * _IDX_CHUNK, _IDX_CHUNK)], idxc_v
        )
        for g in range(_IDX_CHUNK // 16):
            vec = plsc.load_gather(idxc_v, [lane + g * 16, zero16])
            idx_v[pl.ds(c * _IDX_CHUNK + g * 16, 16)] = vec

    def issue_group(g):
        vec = idx_v[pl.ds(g * 16, 16)]
        for l in range(16):
            pltpu.async_copy(
                table_hbm.at[pl.ds(vec[l], 1)],
                rows_v.at[pl.ds(g * 16 + l, 1)],
                sem,
            )

    n_groups = _B_PER_W // 16

    def body(g, _):
        issue_group(g)
        return ()

    lax.fori_loop(0, n_groups, body, ())
    pltpu.make_async_copy(
        table_hbm.at[pl.ds(0, _B_PER_W)], rows_v, sem
    ).wait()

    c1 = pltpu.async_copy(rows_v, out_hbm.at[pl.ds(base, _B_PER_W)], sem)
    c2 = pltpu.async_copy(rows_v, out2_hbm.at[pl.ds(base, _B_PER_W), 0], sem2)
    c1.wait()
    c2.wait()


def kernel(input_ids, table):
    vec, hid = _gather_rows(table, input_ids.astype(jnp.int32))
    return (vec, hid)


# speedup vs baseline: 1.8268x; 1.8268x over previous
import functools

import jax
import jax.numpy as jnp
from jax import lax
from jax.experimental import pallas as pl
from jax.experimental.pallas import tpu as pltpu, tpu_sc as plsc

NUM_SENTENCES = 1000000
EMBED_DIM = 64
BATCH = 16384

_info = plsc.get_sparse_core_info()
_NC, _NS = _info.num_cores, _info.num_subcores
_NW = _NC * _NS
_B_PER_W = BATCH // _NW
_IDX_CHUNK = 128
_NSLOT = 8

_mesh = plsc.VectorSubcoreMesh(core_axis_name="c", subcore_axis_name="s")

_slot_scratch = [pltpu.VMEM((8, 8, 128), jnp.float32) for _ in range(_NSLOT)]
_sem_scratch = [pltpu.SemaphoreType.DMA for _ in range(_NSLOT + 2)]


@functools.partial(
    pl.kernel,
    mesh=_mesh,
    out_type=(
        jax.ShapeDtypeStruct((EMBED_DIM, BATCH), jnp.float32),
        jax.ShapeDtypeStruct((EMBED_DIM, BATCH), jnp.float32),
    ),
    scratch_types=[
        pltpu.VMEM((_B_PER_W,), jnp.int32),
        pltpu.VMEM((_IDX_CHUNK, 1), jnp.int32),
        pltpu.VMEM((EMBED_DIM, _B_PER_W), jnp.float32),
    ]
    + _slot_scratch
    + _sem_scratch,
    compiler_params=pltpu.CompilerParams(needs_layout_passes=False),
)
def _gather_cols(tbl3_hbm, idx_hbm, out_hbm, out2_hbm, idx_v, idxc_v, rows_v,
                 *slots_and_sems):
    slots = slots_and_sems[:_NSLOT]
    sems = slots_and_sems[_NSLOT:2 * _NSLOT]
    so1, so2 = slots_and_sems[2 * _NSLOT:]
    wid = lax.axis_index("s") * _NC + lax.axis_index("c")
    base = wid * _B_PER_W

    lane = lax.iota(jnp.int32, 16)
    zero16 = jnp.zeros((16,), jnp.int32)
    for c in range(_B_PER_W // _IDX_CHUNK):
        pltpu.sync_copy(
            idx_hbm.at[pl.ds(base + c * _IDX_CHUNK, _IDX_CHUNK)], idxc_v
        )
        for g in range(_IDX_CHUNK // 16):
            vec = plsc.load_gather(idxc_v, [lane + g * 16, zero16])
            idx_v[pl.ds(c * _IDX_CHUNK + g * 16, 16)] = vec

    bms = [m * 16 + lane for m in range(4)]
    rms = [lax.shift_right_logical(b, 3) for b in bms]
    rrms = [lax.bitwise_and(b, 7) for b in bms]

    def issue(v, s):
        cs = pl.multiple_of(lax.shift_right_logical(v, 7) * 128, 128)
        pltpu.async_copy(
            tbl3_hbm.at[pl.ds(0, 8), pl.ds(0, 8), pl.ds(cs, 128)],
            slots[s],
            sems[s],
        )

    def wait(s):
        pltpu.make_async_copy(
            tbl3_hbm.at[pl.ds(0, 8), pl.ds(0, 8), pl.ds(0, 128)],
            slots[s],
            sems[s],
        ).wait()

    def extract(w, j, s):
        spl = jnp.broadcast_to(lax.bitwise_and(w, 127), (16,))
        jspl = jnp.broadcast_to(jnp.int32(j), (16,))
        for m in range(4):
            val = plsc.load_gather(slots[s], [rms[m], rrms[m], spl])
            plsc.store_scatter(rows_v, [bms[m], jspl], val)

    vec0 = idx_v[pl.ds(0, 16)]
    for l in range(8):
        issue(vec0[l], l)
    for l in range(8, 16):
        s = l - 8
        wait(s)
        extract(vec0[s], s, s)
        issue(vec0[l], s)

    def body(g, vec_prev):
        vec = idx_v[pl.ds(g * 16, 16)]
        for l in range(8):
            wait(l)
            extract(vec_prev[8 + l], g * 16 + l - 8, l)
            issue(vec[l], l)
        for l in range(8, 16):
            s = l - 8
            wait(s)
            extract(vec[s], g * 16 + l - 8, s)
            issue(vec[l], s)
        return vec

    vec_last = lax.fori_loop(1, _B_PER_W // 16, body, vec0)
    for l in range(8):
        wait(l)
        extract(vec_last[8 + l], _B_PER_W - 8 + l, l)

    c1 = pltpu.async_copy(
        rows_v, out_hbm.at[pl.ds(0, EMBED_DIM), pl.ds(base, _B_PER_W)], so1
    )
    c2 = pltpu.async_copy(
        rows_v, out2_hbm.at[pl.ds(0, EMBED_DIM), pl.ds(base, _B_PER_W)], so2
    )
    c1.wait()
    c2.wait()


def kernel(input_ids, table):
    tbl3 = table.T.reshape(8, 8, NUM_SENTENCES)
    out_t, out2_t = _gather_cols(tbl3, input_ids.astype(jnp.int32))
    return (out_t.T, out2_t.T[:, None, :])


# speedup vs baseline: 1.8733x; 1.0255x over previous
import functools

import jax
import jax.numpy as jnp
from jax import lax
from jax.experimental import pallas as pl
from jax.experimental.pallas import tpu as pltpu, tpu_sc as plsc

NUM_SENTENCES = 1000000
EMBED_DIM = 64
BATCH = 16384

_info = plsc.get_sparse_core_info()
_NC, _NS = _info.num_cores, _info.num_subcores
_NW = _NC * _NS
_B_PER_W = BATCH // _NW
_IDX_CHUNK = 128
_NSLOT = 8

_mesh = plsc.VectorSubcoreMesh(core_axis_name="c", subcore_axis_name="s")

_slot_scratch = [pltpu.VMEM((8, 8, 128), jnp.float32) for _ in range(_NSLOT)]
_sem_scratch = [pltpu.SemaphoreType.DMA for _ in range(_NSLOT + 2)]


@functools.partial(
    pl.kernel,
    mesh=_mesh,
    out_type=(
        jax.ShapeDtypeStruct((EMBED_DIM, BATCH), jnp.float32),
        jax.ShapeDtypeStruct((EMBED_DIM, BATCH), jnp.float32),
    ),
    scratch_types=[
        pltpu.VMEM((_B_PER_W,), jnp.int32),
        pltpu.VMEM((EMBED_DIM, _B_PER_W), jnp.float32),
    ]
    + _slot_scratch
    + _sem_scratch,
    compiler_params=pltpu.CompilerParams(needs_layout_passes=False),
)
def _gather_cols(tbl3_hbm, idx_hbm, out_hbm, out2_hbm, idx_v, rows_v,
                 *slots_and_sems):
    slots = slots_and_sems[:_NSLOT]
    sems = slots_and_sems[_NSLOT:2 * _NSLOT]
    so1, so2 = slots_and_sems[2 * _NSLOT:]
    wid = lax.axis_index("s") * _NC + lax.axis_index("c")
    base = wid * _B_PER_W

    pltpu.sync_copy(idx_hbm.at[pl.ds(base, _B_PER_W)], idx_v)
    lane = lax.iota(jnp.int32, 16)

    bms = [m * 16 + lane for m in range(4)]
    rms = [lax.shift_right_logical(b, 3) for b in bms]
    rrms = [lax.bitwise_and(b, 7) for b in bms]

    def issue(v, s):
        cs = pl.multiple_of(lax.shift_right_logical(v, 7) * 128, 128)
        pltpu.async_copy(
            tbl3_hbm.at[pl.ds(0, 8), pl.ds(0, 8), pl.ds(cs, 128)],
            slots[s],
            sems[s],
        )

    def wait(s):
        pltpu.make_async_copy(
            tbl3_hbm.at[pl.ds(0, 8), pl.ds(0, 8), pl.ds(0, 128)],
            slots[s],
            sems[s],
        ).wait()

    def extract(w, j, s):
        spl = jnp.broadcast_to(lax.bitwise_and(w, 127), (16,))
        jspl = jnp.broadcast_to(jnp.int32(j), (16,))
        for m in range(4):
            val = plsc.load_gather(slots[s], [rms[m], rrms[m], spl])
            plsc.store_scatter(rows_v, [bms[m], jspl], val)

    vec0 = idx_v[pl.ds(0, 16)]
    for l in range(8):
        issue(vec0[l], l)
    for l in range(8, 16):
        s = l - 8
        wait(s)
        extract(vec0[s], s, s)
        issue(vec0[l], s)

    def body(g, vec_prev):
        vec = idx_v[pl.ds(g * 16, 16)]
        for l in range(8):
            wait(l)
            extract(vec_prev[8 + l], g * 16 + l - 8, l)
            issue(vec[l], l)
        for l in range(8, 16):
            s = l - 8
            wait(s)
            extract(vec[s], g * 16 + l - 8, s)
            issue(vec[l], s)
        return vec

    vec_last = lax.fori_loop(1, _B_PER_W // 16, body, vec0)
    for l in range(8):
        wait(l)
        extract(vec_last[8 + l], _B_PER_W - 8 + l, l)

    c1 = pltpu.async_copy(
        rows_v, out_hbm.at[pl.ds(0, EMBED_DIM), pl.ds(base, _B_PER_W)], so1
    )
    c2 = pltpu.async_copy(
        rows_v, out2_hbm.at[pl.ds(0, EMBED_DIM), pl.ds(base, _B_PER_W)], so2
    )
    c1.wait()
    c2.wait()


def kernel(input_ids, table):
    tbl3 = table.T.reshape(8, 8, NUM_SENTENCES)
    idx = jnp.reshape(input_ids.astype(jnp.int32), (-1,))
    out_t, out2_t = _gather_cols(tbl3, idx)
    return (out_t.T, out2_t.T[:, None, :])
